# baseline (speedup 1.0000x reference)
import functools

import jax
import jax.numpy as jnp
from jax import lax
from jax.experimental import pallas as pl
from jax.experimental.pallas import tpu as pltpu
from jax.experimental.pallas import tpu_sc as plsc

N_VOCAB = 49408
D_MODEL = 768
SEQ_LEN = 77
BATCH = 1024
SEQ_PAD = 80
DTH = D_MODEL // 3

_info = plsc.get_sparse_core_info()
_NC = _info.num_cores
_NS = _info.num_subcores
_NW = _NC * _NS
_BPW = BATCH // _NW

_mesh = plsc.VectorSubcoreMesh(core_axis_name="c", subcore_axis_name="s")


@functools.partial(
    pl.kernel,
    mesh=_mesh,
    out_type=jax.ShapeDtypeStruct((BATCH, SEQ_LEN, D_MODEL), jnp.float32),
    scratch_types=[
        pltpu.VMEM((_BPW * SEQ_PAD,), jnp.int32),
        pltpu.VMEM((SEQ_LEN, DTH), jnp.float32),
        pltpu.VMEM((SEQ_LEN, DTH), jnp.float32),
        pltpu.VMEM((SEQ_LEN, DTH), jnp.float32),
        pltpu.VMEM((SEQ_LEN, D_MODEL), jnp.float32),
        pltpu.SemaphoreType.DMA,
        pltpu.SemaphoreType.DMA,
        pltpu.SemaphoreType.DMA,
        pltpu.SemaphoreType.DMA,
        pltpu.SemaphoreType.DMA,
        pltpu.SemaphoreType.DMA,
    ],
)
def _clip_embed(tok_hbm, table_hbm, pos_hbm, out_hbm,
                idx_v, rb0, rb1, rb2, pos_v,
                gs0, gs1, gs2, ws0, ws1, ws2):
    wid = lax.axis_index("s") * _NC + lax.axis_index("c")
    b0 = wid * _BPW
    bufs = (rb0, rb1, rb2)
    gsems = (gs0, gs1, gs2)
    wsems = (ws0, ws1, ws2)

    pltpu.sync_copy(tok_hbm.at[pl.ds(b0 * SEQ_PAD, _BPW * SEQ_PAD)], idx_v)
    pltpu.sync_copy(pos_hbm, pos_v)

    def gather_start(bl, c):
        pltpu.make_async_copy(
            table_hbm.at[
                idx_v.at[pl.ds(bl * SEQ_PAD, SEQ_LEN)],
                pl.ds(c * DTH, DTH),
            ],
            bufs[c],
            gsems[c],
        ).start()

    def gather_wait(c):
        pltpu.make_async_copy(
            table_hbm.at[idx_v.at[pl.ds(0, SEQ_LEN)], pl.ds(0, DTH)],
            bufs[c],
            gsems[c],
        ).wait()

    def write_start(bl, c):
        pltpu.make_async_copy(
            bufs[c],
            out_hbm.at[b0 + bl, slice(None), pl.ds(c * DTH, DTH)],
            wsems[c],
        ).start()

    def write_wait(c):
        pltpu.make_async_copy(
            bufs[c],
            out_hbm.at[0, slice(None), pl.ds(0, DTH)],
            wsems[c],
        ).wait()

    def compute(c):
        buf = bufs[c]

        def row_body(r, acc):
            for dblk in range(DTH // 16):
                sl = pl.ds(dblk * 16, 16)
                buf[r, sl] = buf[r, sl] + pos_v[r, pl.ds(c * DTH + dblk * 16, 16)]
            return acc

        lax.fori_loop(0, SEQ_LEN, row_body, 0)

    def sync_body(bl, carry):
        for c in range(3):
            gather_start(bl, c)
            gather_wait(c)
            compute(c)
            write_start(bl, c)
            write_wait(c)
        return carry

    lax.fori_loop(0, _BPW, sync_body, 0)
    return

    gather_start(0, 0)
    gather_wait(0)
    gather_start(0, 1)
    compute(0)
    write_start(0, 0)
    gather_wait(1)
    gather_start(0, 2)
    compute(1)
    write_start(0, 1)
    gather_wait(2)
    write_wait(0)
    gather_start(1, 0)
    compute(2)
    write_start(0, 2)

    def batch_body(bl, carry):
        for c in range(3):
            gather_wait(c)
            write_wait((c + 1) % 3)
            nb = bl + (c + 1) // 3
            gather_start(nb, (c + 1) % 3)
            compute(c)
            write_start(bl, c)
        return carry

    lax.fori_loop(1, _BPW - 1, batch_body, 0)

    bl = _BPW - 1
    gather_wait(0)
    write_wait(1)
    gather_start(bl, 1)
    compute(0)
    write_start(bl, 0)
    gather_wait(1)
    write_wait(2)
    gather_start(bl, 2)
    compute(1)
    write_start(bl, 1)
    gather_wait(2)
    write_wait(0)
    compute(2)
    write_start(bl, 2)
    write_wait(1)
    write_wait(2)


def kernel(tokens, tok_table, pos_table):
    tok_pad = jnp.pad(tokens, ((0, 0), (0, SEQ_PAD - SEQ_LEN))).reshape(-1)
    return _clip_embed(tok_pad, tok_table, pos_table)


# speedup vs baseline: 1.0529x; 1.0529x over previous
import functools

import jax
import jax.numpy as jnp
from jax import lax
from jax.experimental import pallas as pl
from jax.experimental.pallas import tpu as pltpu
from jax.experimental.pallas import tpu_sc as plsc

N_VOCAB = 49408
D_MODEL = 768
SEQ_LEN = 77
BATCH = 1024
ROWS = BATCH * SEQ_LEN

_info = plsc.get_sparse_core_info()
_NC = _info.num_cores
_NS = _info.num_subcores
_NW = _NC * _NS
_RPW = ROWS // _NW
_CH = 16
_NCHUNK = _RPW // _CH

_mesh = plsc.VectorSubcoreMesh(core_axis_name="c", subcore_axis_name="s")


@functools.partial(
    pl.kernel,
    mesh=_mesh,
    out_type=jax.ShapeDtypeStruct((ROWS, D_MODEL), jnp.float32),
    scratch_types=[
        pltpu.VMEM((_RPW,), jnp.int32),
        pltpu.VMEM((_CH, D_MODEL), jnp.float32),
        pltpu.VMEM((_CH, D_MODEL), jnp.float32),
        pltpu.VMEM((_CH, D_MODEL), jnp.float32),
        pltpu.VMEM((_CH, D_MODEL), jnp.float32),
        pltpu.VMEM((SEQ_LEN, D_MODEL), jnp.float32),
        pltpu.SemaphoreType.DMA,
        pltpu.SemaphoreType.DMA,
        pltpu.SemaphoreType.DMA,
        pltpu.SemaphoreType.DMA,
    ],
)
def _clip_embed(tok_hbm, table_hbm, pos_hbm, out_hbm,
                idx_v, gb0, gb1, ob0, ob1, pos_v,
                gs0, gs1, ws0, ws1):
    wid = lax.axis_index("s") * _NC + lax.axis_index("c")
    wbase = wid * _RPW
    gbufs = (gb0, gb1)
    obufs = (ob0, ob1)
    gsems = (gs0, gs1)
    wsems = (ws0, ws1)

    pltpu.sync_copy(tok_hbm.at[pl.ds(wbase, _RPW)], idx_v)
    pltpu.sync_copy(pos_hbm, pos_v)

    def gather_start(c, par):
        off = pl.multiple_of(c * _CH, _CH)
        pltpu.make_async_copy(
            table_hbm.at[idx_v.at[pl.ds(off, _CH)]], gbufs[par], gsems[par]
        ).start()

    def gather_wait(par):
        pltpu.make_async_copy(
            table_hbm.at[idx_v.at[pl.ds(0, _CH)]], gbufs[par], gsems[par]
        ).wait()

    def write_start(c, par):
        off = pl.multiple_of(wbase + c * _CH, _CH)
        pltpu.make_async_copy(
            obufs[par], out_hbm.at[pl.ds(off, _CH)], wsems[par]
        ).start()

    def write_wait(par):
        pltpu.make_async_copy(
            obufs[par], out_hbm.at[pl.ds(0, _CH)], wsems[par]
        ).wait()

    def compute(c, par):
        p0 = lax.rem(c * _CH, SEQ_LEN)
        gb, ob = gbufs[par], obufs[par]

        def row_body(r, acc):
            p = p0 + r
            p = jnp.where(p >= SEQ_LEN, p - SEQ_LEN, p)
            for dblk in range(D_MODEL // 16):
                sl = pl.ds(dblk * 16, 16)
                ob[r, sl] = gb[r, sl] + pos_v[p, sl]
            return acc

        lax.fori_loop(0, _CH, row_body, 0)

    gather_start(0, 0)
    gather_start(1, 1)
    for c in (0, 1):
        gather_wait(c)
        compute(c, c)
        gather_start(c + 2, c)
        write_start(c, c)

    def pair_body(k, carry):
        for par in (0, 1):
            c = 2 * k + par
            gather_wait(par)
            write_wait(par)
            compute(c, par)
            gather_start(c + 2, par)
            write_start(c, par)
        return carry

    lax.fori_loop(1, _NCHUNK // 2 - 1, pair_body, 0)

    for c in (_NCHUNK - 2, _NCHUNK - 1):
        par = c % 2
        gather_wait(par)
        write_wait(par)
        compute(c, par)
        write_start(c, par)
    write_wait(0)
    write_wait(1)


def kernel(tokens, tok_table, pos_table):
    out = _clip_embed(tokens.reshape(-1), tok_table, pos_table)
    return out.reshape(BATCH, SEQ_LEN, D_MODEL)
